# Initial kernel scaffold; baseline (speedup 1.0000x reference)
import jax
import jax.numpy as jnp
from jax.experimental import pallas as pl


def kernel(pos_ids, ner_ids, table_pos, table_ner):
    raise NotImplementedError("write your pallas kernel here")


# speedup vs baseline: 7.4715x; 7.4715x over previous
import functools

import jax
import jax.numpy as jnp
from jax import lax
from jax.experimental import pallas as pl
from jax.experimental.pallas import tpu as pltpu
from jax.experimental.pallas import tpu_sc as plsc


def _build(N, D):
    info = plsc.get_sparse_core_info()
    NC, NS = info.num_cores, info.num_subcores
    NW = NC * NS
    G = 128
    ROWS = N // G
    ROWS_PER_W = ROWS // NW
    CROWS = 8
    NCHUNK = ROWS_PER_W // CROWS
    CH = CROWS * G

    mesh = plsc.VectorSubcoreMesh(core_axis_name="c", subcore_axis_name="s")

    @functools.partial(
        pl.kernel,
        out_type=jax.ShapeDtypeStruct((N, 2, D), jnp.float32),
        mesh=mesh,
        scratch_types=[
            pltpu.VMEM((CROWS, G), jnp.int32),
            pltpu.VMEM((CROWS, G), jnp.int32),
            pltpu.VMEM((CH, D), jnp.float32),
            pltpu.VMEM((CH, D), jnp.float32),
            pltpu.SemaphoreType.DMA,
        ],
        compiler_params=pltpu.CompilerParams(use_tc_tiling_on_sc=False),
    )
    def emb(pos_hbm, ner_hbm, tpos_hbm, tner_hbm, out_hbm,
            idx_p, idx_n, rows_p, rows_n, sem):
        wid = lax.axis_index("s") * NC + lax.axis_index("c")

        def chunk(i, carry):
            r0 = wid * ROWS_PER_W + i * CROWS
            b0 = r0 * G
            pltpu.sync_copy(pos_hbm.at[pl.ds(r0, CROWS)], idx_p)
            pltpu.sync_copy(ner_hbm.at[pl.ds(r0, CROWS)], idx_n)
            handles = []
            for j in range(CROWS):
                handles.append(pltpu.async_copy(
                    tpos_hbm.at[idx_p.at[j]],
                    rows_p.at[pl.ds(j * G, G)], sem))
                handles.append(pltpu.async_copy(
                    tner_hbm.at[idx_n.at[j]],
                    rows_n.at[pl.ds(j * G, G)], sem))
            for h in handles:
                h.wait()
            pltpu.sync_copy(rows_p, out_hbm.at[pl.ds(b0, CH), 0])
            pltpu.sync_copy(rows_n, out_hbm.at[pl.ds(b0, CH), 1])
            return carry

        lax.fori_loop(0, NCHUNK, chunk, 0)

    return emb


@jax.jit
def kernel(pos_ids, ner_ids, table_pos, table_ner):
    B, L = pos_ids.shape
    V, D = table_pos.shape
    N = B * L
    G = 128
    pos2 = pos_ids.reshape(N // G, G).astype(jnp.int32)
    ner2 = ner_ids.reshape(N // G, G).astype(jnp.int32)
    out3 = _build(N, D)(pos2, ner2, table_pos, table_ner)
    return out3.reshape(B, L, 2 * D)


# speedup vs baseline: 7.7977x; 1.0437x over previous
import functools

import jax
import jax.numpy as jnp
from jax import lax
from jax.experimental import pallas as pl
from jax.experimental.pallas import tpu as pltpu
from jax.experimental.pallas import tpu_sc as plsc


def _build(N, D):
    info = plsc.get_sparse_core_info()
    NC, NS = info.num_cores, info.num_subcores
    NW = NC * NS
    G = 128
    ROWS = N // G
    ROWS_PER_W = ROWS // NW
    CROWS = 4
    NCHUNK = ROWS_PER_W // CROWS
    CH = CROWS * G

    mesh = plsc.VectorSubcoreMesh(core_axis_name="c", subcore_axis_name="s")

    @functools.partial(
        pl.kernel,
        out_type=jax.ShapeDtypeStruct((N, 2, D), jnp.float32),
        mesh=mesh,
        scratch_types=[
            pltpu.VMEM((ROWS_PER_W, G), jnp.int32),
            pltpu.VMEM((ROWS_PER_W, G), jnp.int32),
            pltpu.VMEM((2, CH, D), jnp.float32),
            pltpu.VMEM((2, CH, D), jnp.float32),
            pltpu.SemaphoreType.DMA((2,)),
            pltpu.SemaphoreType.DMA((2,)),
        ],
        compiler_params=pltpu.CompilerParams(use_tc_tiling_on_sc=False),
    )
    def emb(pos_hbm, ner_hbm, tpos_hbm, tner_hbm, out_hbm,
            idx_p, idx_n, rows_p, rows_n, gsem, wsem):
        wid = lax.axis_index("s") * NC + lax.axis_index("c")
        row0 = wid * ROWS_PER_W

        pltpu.sync_copy(pos_hbm.at[pl.ds(row0, ROWS_PER_W)], idx_p)
        pltpu.sync_copy(ner_hbm.at[pl.ds(row0, ROWS_PER_W)], idx_n)

        def fire_gathers(c, b):
            for j in range(CROWS):
                r = c * CROWS + j
                pltpu.async_copy(tpos_hbm.at[idx_p.at[r]],
                                 rows_p.at[b, pl.ds(j * G, G)], gsem.at[b])
                pltpu.async_copy(tner_hbm.at[idx_n.at[r]],
                                 rows_n.at[b, pl.ds(j * G, G)], gsem.at[b])

        def drain_gathers(b):
            for j in range(CROWS):
                pltpu.make_async_copy(tpos_hbm.at[idx_p.at[0]],
                                      rows_p.at[b, pl.ds(0, G)], gsem.at[b]).wait()
                pltpu.make_async_copy(tner_hbm.at[idx_n.at[0]],
                                      rows_n.at[b, pl.ds(0, G)], gsem.at[b]).wait()

        def fire_writes(c, b):
            b0 = (row0 + c * CROWS) * G
            pltpu.async_copy(rows_p.at[b], out_hbm.at[pl.ds(b0, CH), 0], wsem.at[b])
            pltpu.async_copy(rows_n.at[b], out_hbm.at[pl.ds(b0, CH), 1], wsem.at[b])

        def drain_writes(b):
            pltpu.make_async_copy(rows_p.at[b], out_hbm.at[pl.ds(0, CH), 0],
                                  wsem.at[b]).wait()
            pltpu.make_async_copy(rows_n.at[b], out_hbm.at[pl.ds(0, CH), 1],
                                  wsem.at[b]).wait()

        fire_gathers(0, 0)
        fire_gathers(1, 1)

        def pair(cc, carry):
            for b in range(2):
                c = 2 * cc + b
                drain_gathers(b)
                fire_writes(c, b)
                drain_writes(b)
                @pl.when(c + 2 < NCHUNK)
                def _():
                    fire_gathers(c + 2, b)
            return carry

        pl.loop(0, NCHUNK // 2)(lambda cc: pair(cc, None))

    return emb


@jax.jit
def kernel(pos_ids, ner_ids, table_pos, table_ner):
    B, L = pos_ids.shape
    V, D = table_pos.shape
    N = B * L
    G = 128
    pos2 = pos_ids.reshape(N // G, G).astype(jnp.int32)
    ner2 = ner_ids.reshape(N // G, G).astype(jnp.int32)
    out3 = _build(N, D)(pos2, ner2, table_pos, table_ner)
    return out3.reshape(B, L, 2 * D)


# speedup vs baseline: 7.8157x; 1.0023x over previous
import functools

import jax
import jax.numpy as jnp
from jax import lax
from jax.experimental import pallas as pl
from jax.experimental.pallas import tpu as pltpu
from jax.experimental.pallas import tpu_sc as plsc


def _build(B, L, D):
    info = plsc.get_sparse_core_info()
    NC, NS = info.num_cores, info.num_subcores
    NW = NC * NS
    BW = B // NW
    CB = 2
    NCHUNK = BW // CB
    G1 = 104
    G2 = L - G1

    mesh = plsc.VectorSubcoreMesh(core_axis_name="c", subcore_axis_name="s")

    @functools.partial(
        pl.kernel,
        out_type=jax.ShapeDtypeStruct((B, L, 2 * D), jnp.float32),
        mesh=mesh,
        scratch_types=[
            pltpu.VMEM((BW, L), jnp.int32),
            pltpu.VMEM((BW, L), jnp.int32),
            pltpu.VMEM((2, CB, L, D), jnp.float32),
            pltpu.VMEM((2, CB, L, D), jnp.float32),
            pltpu.SemaphoreType.DMA((2,)),
            pltpu.SemaphoreType.DMA((2,)),
        ],
        compiler_params=pltpu.CompilerParams(use_tc_tiling_on_sc=False),
    )
    def emb(pos_hbm, ner_hbm, tpos_hbm, tner_hbm, out_hbm,
            idx_p, idx_n, rows_p, rows_n, gsem, wsem):
        wid = lax.axis_index("s") * NC + lax.axis_index("c")
        b0 = wid * BW

        pltpu.sync_copy(pos_hbm.at[pl.ds(b0, BW)], idx_p)
        pltpu.sync_copy(ner_hbm.at[pl.ds(b0, BW)], idx_n)

        def fire_gathers(c, b):
            for rl in range(CB):
                r = c * CB + rl
                for off, g in ((0, G1), (G1, G2)):
                    pltpu.async_copy(tpos_hbm.at[idx_p.at[r, pl.ds(off, g)]],
                                     rows_p.at[b, rl, pl.ds(off, g)], gsem.at[b])
                    pltpu.async_copy(tner_hbm.at[idx_n.at[r, pl.ds(off, g)]],
                                     rows_n.at[b, rl, pl.ds(off, g)], gsem.at[b])

        def drain_gathers(b):
            for rl in range(CB):
                for off, g in ((0, G1), (G1, G2)):
                    pltpu.make_async_copy(
                        tpos_hbm.at[idx_p.at[0, pl.ds(off, g)]],
                        rows_p.at[b, 0, pl.ds(off, g)], gsem.at[b]).wait()
                    pltpu.make_async_copy(
                        tner_hbm.at[idx_n.at[0, pl.ds(off, g)]],
                        rows_n.at[b, 0, pl.ds(off, g)], gsem.at[b]).wait()

        def fire_writes(c, b):
            bg = b0 + c * CB
            pltpu.async_copy(rows_p.at[b],
                             out_hbm.at[pl.ds(bg, CB), :, pl.ds(0, D)],
                             wsem.at[b])
            pltpu.async_copy(rows_n.at[b],
                             out_hbm.at[pl.ds(bg, CB), :, pl.ds(D, D)],
                             wsem.at[b])

        def drain_writes(b):
            pltpu.make_async_copy(rows_p.at[b],
                                  out_hbm.at[pl.ds(0, CB), :, pl.ds(0, D)],
                                  wsem.at[b]).wait()
            pltpu.make_async_copy(rows_n.at[b],
                                  out_hbm.at[pl.ds(0, CB), :, pl.ds(D, D)],
                                  wsem.at[b]).wait()

        fire_gathers(0, 0)
        fire_gathers(1, 1)

        def pair(cc, carry):
            for b in range(2):
                c = 2 * cc + b
                drain_gathers(b)
                fire_writes(c, b)
                drain_writes(b)
                @pl.when(c + 2 < NCHUNK)
                def _():
                    fire_gathers(c + 2, b)
            return carry

        pl.loop(0, NCHUNK // 2)(lambda cc: pair(cc, None))

    return emb


@jax.jit
def kernel(pos_ids, ner_ids, table_pos, table_ner):
    B, L = pos_ids.shape
    V, D = table_pos.shape
    return _build(B, L, D)(pos_ids, ner_ids, table_pos, table_ner)
